# baseline (device time: 40529 ns/iter reference)
import jax
import jax.numpy as jnp
from jax import lax
from jax.experimental import pallas as pl
from jax.experimental.pallas import tpu as pltpu

N_DEV = 4
SQ = 1024
SKV = 1024
D_MODEL = 1024
H_PER_SHARD = 8
DH = 128
SCALE = 0.08838834764831843
N_GROUPS = 4
GQ = SQ // N_GROUPS
GK = SKV // N_GROUPS
BLK = 64
CHUNK = SQ // 2 // N_DEV


def _perm_rows(a):
    n, c = a.shape
    return a.reshape(N_GROUPS, N_GROUPS, n // 16, c).transpose(1, 0, 2, 3).reshape(n, c)


def kernel(x, Wq, K_ext, V_ext, Wo):

    def body(x_hbm, wq_hbm, kext_ref, vext_ref, wo_hbm, out_ref,
             x_ref, w_f, wq16, wo16, kscr, vscr,
             ctx_ref, part_ref,
             stage8, stage_sc, rs8, rs_sc, ag_stage8, ag8, ag_sc,
             kv_sems, in_sems, out_sems, d_send, d_recv, s_send, s_recv,
             ag_send, ag_recv, ags_send, ags_recv):
        my = lax.axis_index("i")

        barrier_sem = pltpu.get_barrier_semaphore()
        for k in range(1, N_DEV):
            pl.semaphore_signal(
                barrier_sem, inc=1,
                device_id=(lax.rem(my + k, N_DEV),),
                device_id_type=pl.DeviceIdType.MESH,
            )
        pl.semaphore_wait(barrier_sem, N_DEV - 1)

        h0 = my * H_PER_SHARD
        wqcopy = pltpu.make_async_copy(wq_hbm, w_f, in_sems.at[0])
        xcopy = pltpu.make_async_copy(
            x_hbm.at[0, :, :], x_ref, in_sems.at[1]
        )
        kcopy = pltpu.make_async_copy(
            kext_ref.at[0, :, pl.ds(h0, H_PER_SHARD), :], kscr, kv_sems.at[0]
        )
        vcopy = pltpu.make_async_copy(
            vext_ref.at[0, :, pl.ds(h0, H_PER_SHARD), :], vscr, kv_sems.at[1]
        )
        wqcopy.start()
        xcopy.start()
        kcopy.start()
        vcopy.start()
        wqcopy.wait()
        wq16[:] = w_f[:].astype(jnp.bfloat16)
        wocopy = pltpu.make_async_copy(wo_hbm, w_f, in_sems.at[2])
        wocopy.start()
        xcopy.wait()
        kcopy.wait()
        vcopy.wait()
        wo_state = {"ready": False}

        def ensure_wo16():
            if not wo_state["ready"]:
                wocopy.wait()
                wo16[:] = w_f[:].astype(jnp.bfloat16)
                wo_state["ready"] = True

        def compute_chunk(row0):
            g = row0 // GQ
            sub = lax.rem(row0 // CHUNK, 2)
            o0 = 2 * sub
            xq = jnp.concatenate(
                [x_ref[pl.ds((o0 + j) * GQ + g * BLK, BLK), :]
                 for j in range(2)], axis=0,
            ).astype(jnp.bfloat16)
            qc = jnp.dot(
                xq, wq16[:], preferred_element_type=jnp.float32
            ).astype(jnp.bfloat16)
            kq = jnp.concatenate(
                [kscr[pl.ds(o * GK + g * BLK, BLK), :, :]
                 for o in range(N_GROUPS)], axis=0,
            ).astype(jnp.bfloat16).reshape(GK, H_PER_SHARD * DH)
            vq = jnp.concatenate(
                [vscr[pl.ds(o * GK + g * BLK, BLK), :, :]
                 for o in range(N_GROUPS)], axis=0,
            ).astype(jnp.bfloat16).reshape(GK, H_PER_SHARD * DH)
            for h in range(H_PER_SHARD):
                kh = kq[:, h * DH:(h + 1) * DH]
                vh = vq[:, h * DH:(h + 1) * DH]
                s = lax.dot_general(
                    qc[:, h * DH:(h + 1) * DH], kh,
                    (((1,), (1,)), ((), ())),
                    preferred_element_type=jnp.float32,
                ) * SCALE
                m = jnp.max(s, axis=1, keepdims=True)
                w = jnp.exp(s - m)
                p = w / jnp.sum(w, axis=1, keepdims=True)
                ctx_ref[:, h * DH:(h + 1) * DH] = jnp.dot(
                    p.astype(jnp.bfloat16), vh,
                    preferred_element_type=jnp.float32,
                ).astype(jnp.bfloat16)
            ensure_wo16()
            part_ref[pl.ds(row0, CHUNK), :] = jnp.dot(
                ctx_ref[:], wo16[:], preferred_element_type=jnp.float32
            )

        def quantize(val, sc_ref):
            m = jnp.maximum(jnp.max(jnp.abs(val)), 1e-20)
            sc_ref[:] = jnp.full((8, 128), m * (1.0 / 127.0), jnp.float32)
            return jnp.clip(
                jnp.round(val * (127.0 / m)), -127.0, 127.0
            ).astype(jnp.int8)

        def start_pair(data_src, data_dst, sc_src, sc_dst, dsem, rsem,
                       ssem, srsem, dest):
            d = pltpu.make_async_remote_copy(
                src_ref=data_src, dst_ref=data_dst,
                send_sem=dsem, recv_sem=rsem,
                device_id=(dest,), device_id_type=pl.DeviceIdType.MESH,
            )
            s = pltpu.make_async_remote_copy(
                src_ref=sc_src, dst_ref=sc_dst,
                send_sem=ssem, recv_sem=srsem,
                device_id=(dest,), device_id_type=pl.DeviceIdType.MESH,
            )
            d.start()
            s.start()
            return d, s

        def qrows(q):
            return pl.ds(q * GQ, GQ)

        rs_rdmas = []
        for k in range(1, N_DEV):
            c = lax.rem(my + k, N_DEV)
            compute_chunk(c * GQ)
            compute_chunk(c * GQ + CHUNK)
            stage8[k - 1] = quantize(
                part_ref[qrows(c), :], stage_sc.at[k - 1]
            )
            rs_rdmas.append(start_pair(
                stage8.at[k - 1], rs8.at[3 - k],
                stage_sc.at[k - 1], rs_sc.at[3 - k],
                d_send.at[k - 1], d_recv.at[3 - k],
                s_send.at[k - 1], s_recv.at[3 - k], c,
            ))

        compute_chunk(my * GQ)
        compute_chunk(my * GQ + CHUNK)
        acc = part_ref[pl.ds(my * GQ, GQ), :]
        for k in range(1, N_DEV):
            d, s = rs_rdmas[k - 1]
            d.wait()
            s.wait()
            acc = acc + (rs_sc[3 - k, 0:1, 0:1]
                         * rs8[3 - k].astype(jnp.float32))

        out_copies = []

        def store_quarter(q, val, slot):
            for o in range(N_GROUPS):
                rs = pl.ds(o * GQ + q * BLK, BLK)
                part_ref[rs, :] = val[o * BLK:(o + 1) * BLK, :]
                cp = pltpu.make_async_copy(
                    part_ref.at[rs, :], out_ref.at[0, rs, :],
                    out_sems.at[slot, o],
                )
                cp.start()
                out_copies.append(cp)

        store_quarter(my, acc, 3)

        ag_stage8[:] = quantize(acc, stage_sc.at[3])
        ag_rdmas = []
        for k in range(1, N_DEV):
            c = lax.rem(my + k, N_DEV)
            ag_rdmas.append(start_pair(
                ag_stage8, ag8.at[3 - k],
                stage_sc.at[3], ag_sc.at[3 - k],
                ag_send.at[k - 1], ag_recv.at[3 - k],
                ags_send.at[k - 1], ags_recv.at[3 - k], c,
            ))
        for k in range(1, N_DEV):
            d, s = ag_rdmas[k - 1]
            d.wait()
            s.wait()
            q = lax.rem(my + N_DEV - k, N_DEV)
            store_quarter(
                q, ag_sc[3 - k, 0:1, 0:1] * ag8[3 - k].astype(jnp.float32),
                k - 1,
            )
        for cp in out_copies:
            cp.wait()

    return pl.pallas_call(
        body,
        out_shape=jax.ShapeDtypeStruct((1, SQ, D_MODEL), jnp.float32),
        in_specs=[
            pl.BlockSpec(memory_space=pltpu.MemorySpace.HBM)
        ] * 5,
        out_specs=pl.BlockSpec(memory_space=pltpu.MemorySpace.HBM),
        scratch_shapes=[
            pltpu.VMEM((SQ, D_MODEL), jnp.float32),
            pltpu.VMEM((D_MODEL, D_MODEL), jnp.float32),
            pltpu.VMEM((D_MODEL, D_MODEL), jnp.bfloat16),
            pltpu.VMEM((D_MODEL, D_MODEL), jnp.bfloat16),
            pltpu.VMEM((SKV, H_PER_SHARD, DH), jnp.float32),
            pltpu.VMEM((SKV, H_PER_SHARD, DH), jnp.float32),
            pltpu.VMEM((CHUNK, H_PER_SHARD * DH), jnp.bfloat16),
            pltpu.VMEM((SQ, D_MODEL), jnp.float32),
            pltpu.VMEM((N_DEV - 1, GQ, D_MODEL), jnp.int8),
            pltpu.VMEM((N_DEV, 8, 128), jnp.float32),
            pltpu.VMEM((N_DEV - 1, GQ, D_MODEL), jnp.int8),
            pltpu.VMEM((N_DEV - 1, 8, 128), jnp.float32),
            pltpu.VMEM((GQ, D_MODEL), jnp.int8),
            pltpu.VMEM((N_DEV - 1, GQ, D_MODEL), jnp.int8),
            pltpu.VMEM((N_DEV - 1, 8, 128), jnp.float32),
            pltpu.SemaphoreType.DMA((2,)),
            pltpu.SemaphoreType.DMA((3,)),
            pltpu.SemaphoreType.DMA((N_DEV, N_GROUPS)),
            pltpu.SemaphoreType.DMA((N_DEV - 1,)),
            pltpu.SemaphoreType.DMA((N_DEV - 1,)),
            pltpu.SemaphoreType.DMA((N_DEV - 1,)),
            pltpu.SemaphoreType.DMA((N_DEV - 1,)),
            pltpu.SemaphoreType.DMA((N_DEV - 1,)),
            pltpu.SemaphoreType.DMA((N_DEV - 1,)),
            pltpu.SemaphoreType.DMA((N_DEV - 1,)),
            pltpu.SemaphoreType.DMA((N_DEV - 1,)),
        ],
        compiler_params=pltpu.CompilerParams(collective_id=0),
    )(x, Wq, K_ext, V_ext, Wo)


# device time: 39816 ns/iter; 1.0179x vs baseline; 1.0179x over previous
import jax
import jax.numpy as jnp
from jax import lax
from jax.experimental import pallas as pl
from jax.experimental.pallas import tpu as pltpu

N_DEV = 4
SQ = 1024
SKV = 1024
D_MODEL = 1024
H_PER_SHARD = 8
DH = 128
SCALE = 0.08838834764831843
N_GROUPS = 4
GQ = SQ // N_GROUPS
GK = SKV // N_GROUPS
BLK = 64
CHUNK = SQ // 2 // N_DEV


def _perm_rows(a):
    n, c = a.shape
    return a.reshape(N_GROUPS, N_GROUPS, n // 16, c).transpose(1, 0, 2, 3).reshape(n, c)


def kernel(x, Wq, K_ext, V_ext, Wo):

    def body(x_hbm, wq_hbm, kext_ref, vext_ref, wo_hbm, out_ref,
             x_ref, w_f, wq16, wo16, kscr, vscr,
             ctx_ref, part_ref,
             stage8, stage_sc, rs8, rs_sc, ag_stage8, ag8, ag_sc,
             kv_sems, in_sems, xg_sems, out_sems,
             d_send, d_recv, s_send, s_recv,
             ag_send, ag_recv, ags_send, ags_recv):
        my = lax.axis_index("i")

        h0 = my * H_PER_SHARD
        wqcopy = pltpu.make_async_copy(wq_hbm, w_f, in_sems.at[0])
        wqcopy.start()
        x_dmas = [[] for _ in range(N_DEV)]
        for p in range(N_DEV):
            g = lax.rem(my + 1 + p, N_DEV)
            for o in range(N_GROUPS):
                rs = pl.ds(o * GQ + g * BLK, BLK)
                cp = pltpu.make_async_copy(
                    x_hbm.at[0, rs, :], x_ref.at[rs, :], xg_sems.at[p, o]
                )
                cp.start()
                x_dmas[p].append(cp)
            if p == 0:
                kcopy = pltpu.make_async_copy(
                    kext_ref.at[0, :, pl.ds(h0, H_PER_SHARD), :],
                    kscr, kv_sems.at[0],
                )
                vcopy = pltpu.make_async_copy(
                    vext_ref.at[0, :, pl.ds(h0, H_PER_SHARD), :],
                    vscr, kv_sems.at[1],
                )
                kcopy.start()
                vcopy.start()
        wocopy = pltpu.make_async_copy(wo_hbm, w_f, in_sems.at[2])

        barrier_sem = pltpu.get_barrier_semaphore()
        for k in range(1, N_DEV):
            pl.semaphore_signal(
                barrier_sem, inc=1,
                device_id=(lax.rem(my + k, N_DEV),),
                device_id_type=pl.DeviceIdType.MESH,
            )
        pl.semaphore_wait(barrier_sem, N_DEV - 1)

        wqcopy.wait()
        wq16[:] = w_f[:].astype(jnp.bfloat16)
        wocopy.start()
        lazy = {"wo": False, "kv": False}

        def ensure_kv():
            if not lazy["kv"]:
                kcopy.wait()
                vcopy.wait()
                lazy["kv"] = True

        def ensure_wo16():
            if not lazy["wo"]:
                wocopy.wait()
                wo16[:] = w_f[:].astype(jnp.bfloat16)
                lazy["wo"] = True

        def compute_chunk(row0):
            g = row0 // GQ
            sub = lax.rem(row0 // CHUNK, 2)
            o0 = 2 * sub
            xq = jnp.concatenate(
                [x_ref[pl.ds((o0 + j) * GQ + g * BLK, BLK), :]
                 for j in range(2)], axis=0,
            ).astype(jnp.bfloat16)
            qc = jnp.dot(
                xq, wq16[:], preferred_element_type=jnp.float32
            ).astype(jnp.bfloat16)
            ensure_kv()
            kq = jnp.concatenate(
                [kscr[pl.ds(o * GK + g * BLK, BLK), :, :]
                 for o in range(N_GROUPS)], axis=0,
            ).astype(jnp.bfloat16).reshape(GK, H_PER_SHARD * DH)
            vq = jnp.concatenate(
                [vscr[pl.ds(o * GK + g * BLK, BLK), :, :]
                 for o in range(N_GROUPS)], axis=0,
            ).astype(jnp.bfloat16).reshape(GK, H_PER_SHARD * DH)
            for h in range(H_PER_SHARD):
                kh = kq[:, h * DH:(h + 1) * DH]
                vh = vq[:, h * DH:(h + 1) * DH]
                s = lax.dot_general(
                    qc[:, h * DH:(h + 1) * DH], kh,
                    (((1,), (1,)), ((), ())),
                    preferred_element_type=jnp.float32,
                ) * SCALE
                m = jnp.max(s, axis=1, keepdims=True)
                w = jnp.exp(s - m)
                p = w / jnp.sum(w, axis=1, keepdims=True)
                ctx_ref[:, h * DH:(h + 1) * DH] = jnp.dot(
                    p.astype(jnp.bfloat16), vh,
                    preferred_element_type=jnp.float32,
                ).astype(jnp.bfloat16)
            ensure_wo16()
            part_ref[pl.ds(row0, CHUNK), :] = jnp.dot(
                ctx_ref[:], wo16[:], preferred_element_type=jnp.float32
            )

        def quantize(val, sc_ref):
            m = jnp.maximum(jnp.max(jnp.abs(val)), 1e-20)
            sc_ref[:] = jnp.full((8, 128), m * (1.0 / 127.0), jnp.float32)
            return jnp.clip(
                jnp.round(val * (127.0 / m)), -127.0, 127.0
            ).astype(jnp.int8)

        def start_pair(data_src, data_dst, sc_src, sc_dst, dsem, rsem,
                       ssem, srsem, dest):
            d = pltpu.make_async_remote_copy(
                src_ref=data_src, dst_ref=data_dst,
                send_sem=dsem, recv_sem=rsem,
                device_id=(dest,), device_id_type=pl.DeviceIdType.MESH,
            )
            s = pltpu.make_async_remote_copy(
                src_ref=sc_src, dst_ref=sc_dst,
                send_sem=ssem, recv_sem=srsem,
                device_id=(dest,), device_id_type=pl.DeviceIdType.MESH,
            )
            d.start()
            s.start()
            return d, s

        def qrows(q):
            return pl.ds(q * GQ, GQ)

        rs_rdmas = []
        for k in range(1, N_DEV):
            c = lax.rem(my + k, N_DEV)
            for cp in x_dmas[k - 1]:
                cp.wait()
            compute_chunk(c * GQ)
            compute_chunk(c * GQ + CHUNK)
            stage8[k - 1] = quantize(
                part_ref[qrows(c), :], stage_sc.at[k - 1]
            )
            rs_rdmas.append(start_pair(
                stage8.at[k - 1], rs8.at[3 - k],
                stage_sc.at[k - 1], rs_sc.at[3 - k],
                d_send.at[k - 1], d_recv.at[3 - k],
                s_send.at[k - 1], s_recv.at[3 - k], c,
            ))

        for cp in x_dmas[N_DEV - 1]:
            cp.wait()
        compute_chunk(my * GQ)
        compute_chunk(my * GQ + CHUNK)
        acc = part_ref[pl.ds(my * GQ, GQ), :]
        for k in range(1, N_DEV):
            d, s = rs_rdmas[k - 1]
            d.wait()
            s.wait()
            acc = acc + (rs_sc[3 - k, 0:1, 0:1]
                         * rs8[3 - k].astype(jnp.float32))

        out_copies = []

        def store_quarter(q, val, slot):
            for o in range(N_GROUPS):
                rs = pl.ds(o * GQ + q * BLK, BLK)
                part_ref[rs, :] = val[o * BLK:(o + 1) * BLK, :]
                cp = pltpu.make_async_copy(
                    part_ref.at[rs, :], out_ref.at[0, rs, :],
                    out_sems.at[slot, o],
                )
                cp.start()
                out_copies.append(cp)

        store_quarter(my, acc, 3)

        ag_stage8[:] = quantize(acc, stage_sc.at[3])
        ag_rdmas = []
        for k in range(1, N_DEV):
            c = lax.rem(my + k, N_DEV)
            ag_rdmas.append(start_pair(
                ag_stage8, ag8.at[3 - k],
                stage_sc.at[3], ag_sc.at[3 - k],
                ag_send.at[k - 1], ag_recv.at[3 - k],
                ags_send.at[k - 1], ags_recv.at[3 - k], c,
            ))
        for k in range(1, N_DEV):
            d, s = ag_rdmas[k - 1]
            d.wait()
            s.wait()
            q = lax.rem(my + N_DEV - k, N_DEV)
            store_quarter(
                q, ag_sc[3 - k, 0:1, 0:1] * ag8[3 - k].astype(jnp.float32),
                k - 1,
            )
        for cp in out_copies:
            cp.wait()

    return pl.pallas_call(
        body,
        out_shape=jax.ShapeDtypeStruct((1, SQ, D_MODEL), jnp.float32),
        in_specs=[
            pl.BlockSpec(memory_space=pltpu.MemorySpace.HBM)
        ] * 5,
        out_specs=pl.BlockSpec(memory_space=pltpu.MemorySpace.HBM),
        scratch_shapes=[
            pltpu.VMEM((SQ, D_MODEL), jnp.float32),
            pltpu.VMEM((D_MODEL, D_MODEL), jnp.float32),
            pltpu.VMEM((D_MODEL, D_MODEL), jnp.bfloat16),
            pltpu.VMEM((D_MODEL, D_MODEL), jnp.bfloat16),
            pltpu.VMEM((SKV, H_PER_SHARD, DH), jnp.float32),
            pltpu.VMEM((SKV, H_PER_SHARD, DH), jnp.float32),
            pltpu.VMEM((CHUNK, H_PER_SHARD * DH), jnp.bfloat16),
            pltpu.VMEM((SQ, D_MODEL), jnp.float32),
            pltpu.VMEM((N_DEV - 1, GQ, D_MODEL), jnp.int8),
            pltpu.VMEM((N_DEV, 8, 128), jnp.float32),
            pltpu.VMEM((N_DEV - 1, GQ, D_MODEL), jnp.int8),
            pltpu.VMEM((N_DEV - 1, 8, 128), jnp.float32),
            pltpu.VMEM((GQ, D_MODEL), jnp.int8),
            pltpu.VMEM((N_DEV - 1, GQ, D_MODEL), jnp.int8),
            pltpu.VMEM((N_DEV - 1, 8, 128), jnp.float32),
            pltpu.SemaphoreType.DMA((2,)),
            pltpu.SemaphoreType.DMA((3,)),
            pltpu.SemaphoreType.DMA((N_DEV, N_GROUPS)),
            pltpu.SemaphoreType.DMA((N_DEV, N_GROUPS)),
            pltpu.SemaphoreType.DMA((N_DEV - 1,)),
            pltpu.SemaphoreType.DMA((N_DEV - 1,)),
            pltpu.SemaphoreType.DMA((N_DEV - 1,)),
            pltpu.SemaphoreType.DMA((N_DEV - 1,)),
            pltpu.SemaphoreType.DMA((N_DEV - 1,)),
            pltpu.SemaphoreType.DMA((N_DEV - 1,)),
            pltpu.SemaphoreType.DMA((N_DEV - 1,)),
            pltpu.SemaphoreType.DMA((N_DEV - 1,)),
        ],
        compiler_params=pltpu.CompilerParams(collective_id=0),
    )(x, Wq, K_ext, V_ext, Wo)
